# baseline (speedup 1.0000x reference)
import functools

import jax
import jax.numpy as jnp
from jax import lax
from jax.experimental import pallas as pl
from jax.experimental.pallas import tpu as pltpu
from jax.experimental.pallas import tpu_sc as plsc

BATCH = 16384
HIST = 200
DIM = 32
HALF = HIST // 2

BBLK = 64

NC, NS, L = 2, 16, 16
NW = NC * NS
ROWS_W = BATCH // NW
SLAB = 8
NSLAB = ROWS_W // SLAB
STRIDE = 208


def _rank_body(ts_ref, rank_ref):
    s = ts_ref[...]
    sj = s[:, :, None]
    sk = s[:, None, :]
    iota_j = lax.broadcasted_iota(jnp.int32, (BBLK, HIST, HIST), 1)
    iota_k = lax.broadcasted_iota(jnp.int32, (BBLK, HIST, HIST), 2)
    cond = (sk < sj) | ((sk == sj) & (iota_k < iota_j))
    cmp = jnp.where(cond, jnp.float32(1.0), jnp.float32(0.0))
    rank_ref[...] = jnp.sum(cmp, axis=2).astype(jnp.int32)


def _ranks(ts):
    return pl.pallas_call(
        _rank_body,
        grid=(BATCH // BBLK,),
        in_specs=[pl.BlockSpec((BBLK, HIST), lambda i: (i, 0))],
        out_specs=pl.BlockSpec((BBLK, HIST), lambda i: (i, 0)),
        out_shape=jax.ShapeDtypeStruct((BATCH, HIST), jnp.int32),
    )(ts)


def _gather_body(tab_hbm, rank_hbm, out_hbm,
                 rank_v0, rank_v1, idxb_v, val_v, ord_v0, ord_v1, ord_sp,
                 buf_v0, buf_v1, gsem, wsem0, wsem1):
    sid = lax.axis_index("s")
    wid = sid * NC + lax.axis_index("c")
    base = wid * ROWS_W
    sp_base = sid * (2 * SLAB * STRIDE)

    for j0 in range(0, SLAB * HIST, L):
        v = lax.broadcasted_iota(jnp.int32, (L,), 0) + j0
        val_v[pl.ds(j0, L)] = lax.rem(v, jnp.int32(HIST))

    def body(g, p):
        row0 = base + g * SLAB
        rank_v = rank_v0 if p == 0 else rank_v1
        ord_v = ord_v0 if p == 0 else ord_v1
        buf_v = buf_v0 if p == 0 else buf_v1
        wsem = wsem0 if p == 0 else wsem1
        pltpu.sync_copy(rank_hbm.at[pl.ds(row0 * HIST, SLAB * HIST)], rank_v)
        pb = sp_base + p * (SLAB * STRIDE)
        for r in range(SLAB):
            for j0 in list(range(0, HIST - L + 1, L)) + [HIST - L]:
                rk = rank_v[pl.ds(r * HIST + j0, L)]
                hi = jnp.where(rk >= HALF, jnp.int32(4), jnp.int32(0))
                idxb_v[pl.ds(r * HIST + j0, L)] = rk + hi + (r * STRIDE + pb)
        pltpu.sync_copy(val_v, ord_sp.at[idxb_v])
        pltpu.sync_copy(ord_sp.at[pl.ds(pb, SLAB * STRIDE)], ord_v)
        @pl.when(g >= 2)
        def _():
            pltpu.make_async_copy(
                buf_v,
                out_hbm.at[pl.ds((row0 - 2 * SLAB) * HIST, SLAB * HIST)],
                wsem).wait()
        copies = []
        for r in range(SLAB):
            for c in range(2):
                copies.append(pltpu.async_copy(
                    tab_hbm.at[ord_v.at[pl.ds(r * STRIDE + c * 104, HALF)]],
                    buf_v.at[pl.ds((2 * r + c) * HALF, HALF)],
                    gsem))
        for cp in copies:
            cp.wait()
        pltpu.async_copy(
            buf_v, out_hbm.at[pl.ds(row0 * HIST, SLAB * HIST)], wsem)

    def two(gg, _):
        body(2 * gg, 0)
        body(2 * gg + 1, 1)
        return 0

    lax.fori_loop(0, NSLAB // 2, two, 0)
    for p, wsem, buf_v in ((0, wsem0, buf_v0), (1, wsem1, buf_v1)):
        pltpu.make_async_copy(
            buf_v,
            out_hbm.at[pl.ds((base + (NSLAB - 2 + p) * SLAB) * HIST,
                             SLAB * HIST)],
            wsem).wait()


def _gather(pos_embeddings, rank_flat):
    mesh = plsc.VectorSubcoreMesh(
        core_axis_name="c", subcore_axis_name="s", num_cores=NC)
    f = functools.partial(
        pl.kernel,
        mesh=mesh,
        compiler_params=pltpu.CompilerParams(use_tc_tiling_on_sc=False),
        out_type=jax.ShapeDtypeStruct((BATCH * HIST, DIM), jnp.float32),
        scratch_types=[
            pltpu.VMEM((SLAB * HIST,), jnp.int32),
            pltpu.VMEM((SLAB * HIST,), jnp.int32),
            pltpu.VMEM((SLAB * HIST,), jnp.int32),
            pltpu.VMEM((SLAB * HIST,), jnp.int32),
            pltpu.VMEM((SLAB * STRIDE,), jnp.int32),
            pltpu.VMEM((SLAB * STRIDE,), jnp.int32),
            pltpu.VMEM_SHARED((NS * 2 * SLAB * STRIDE,), jnp.int32),
            pltpu.VMEM((SLAB * HIST, DIM), jnp.float32),
            pltpu.VMEM((SLAB * HIST, DIM), jnp.float32),
            pltpu.SemaphoreType.DMA,
            pltpu.SemaphoreType.DMA,
            pltpu.SemaphoreType.DMA,
        ],
    )(_gather_body)
    return f(pos_embeddings, rank_flat)


def kernel(ts, pos_embeddings):
    rank = _ranks(ts)
    out = _gather(pos_embeddings, rank.reshape(-1))
    return out.reshape(BATCH, HIST, DIM)


# speedup vs baseline: 1.1023x; 1.1023x over previous
import functools

import jax
import jax.numpy as jnp
from jax import lax
from jax.experimental import pallas as pl
from jax.experimental.pallas import tpu as pltpu
from jax.experimental.pallas import tpu_sc as plsc

BATCH = 16384
HIST = 200
DIM = 32
HALF = HIST // 2

BBLK = 64

NC, NS, L = 2, 16, 16
NW = NC * NS
ROWS_W = BATCH // NW
SLAB = 8
NSLAB = ROWS_W // SLAB
STRIDE = 208


def _rank_body(ts_ref, rank_ref):
    s = ts_ref[...]
    sk = s[:, :, None]
    sj = s[:, None, :]
    i2k = lax.broadcasted_iota(jnp.int32, (HIST, HIST), 0)
    i2j = lax.broadcasted_iota(jnp.int32, (HIST, HIST), 1)
    tri = (i2k < i2j)[None, :, :]
    lt = sk < sj
    le = sk <= sj
    cond = lt | (le & tri)
    cmp = jnp.where(cond, jnp.float32(1.0), jnp.float32(0.0))
    rank_ref[...] = jnp.sum(cmp, axis=1).astype(jnp.int32)


def _ranks(ts):
    return pl.pallas_call(
        _rank_body,
        grid=(BATCH // BBLK,),
        in_specs=[pl.BlockSpec((BBLK, HIST), lambda i: (i, 0))],
        out_specs=pl.BlockSpec((BBLK, HIST), lambda i: (i, 0)),
        out_shape=jax.ShapeDtypeStruct((BATCH, HIST), jnp.int32),
    )(ts)


def _gather_body(tab_hbm, rank_hbm, out_hbm,
                 rank_v0, rank_v1, idxb_v, val_v, ord_v0, ord_v1, ord_sp,
                 buf_v0, buf_v1, gsem, wsem0, wsem1):
    sid = lax.axis_index("s")
    wid = sid * NC + lax.axis_index("c")
    base = wid * ROWS_W
    sp_base = sid * (2 * SLAB * STRIDE)

    for j0 in range(0, SLAB * HIST, L):
        v = lax.broadcasted_iota(jnp.int32, (L,), 0) + j0
        val_v[pl.ds(j0, L)] = lax.rem(v, jnp.int32(HIST))

    def body(g, p):
        row0 = base + g * SLAB
        rank_v = rank_v0 if p == 0 else rank_v1
        ord_v = ord_v0 if p == 0 else ord_v1
        buf_v = buf_v0 if p == 0 else buf_v1
        wsem = wsem0 if p == 0 else wsem1
        pltpu.sync_copy(rank_hbm.at[pl.ds(row0 * HIST, SLAB * HIST)], rank_v)
        pb = sp_base + p * (SLAB * STRIDE)
        for r in range(SLAB):
            for j0 in list(range(0, HIST - L + 1, L)) + [HIST - L]:
                rk = rank_v[pl.ds(r * HIST + j0, L)]
                hi = jnp.where(rk >= HALF, jnp.int32(4), jnp.int32(0))
                idxb_v[pl.ds(r * HIST + j0, L)] = rk + hi + (r * STRIDE + pb)
        pltpu.sync_copy(val_v, ord_sp.at[idxb_v])
        pltpu.sync_copy(ord_sp.at[pl.ds(pb, SLAB * STRIDE)], ord_v)
        @pl.when(g >= 2)
        def _():
            pltpu.make_async_copy(
                buf_v,
                out_hbm.at[pl.ds((row0 - 2 * SLAB) * HIST, SLAB * HIST)],
                wsem).wait()
        copies = []
        for r in range(SLAB):
            for c in range(2):
                copies.append(pltpu.async_copy(
                    tab_hbm.at[ord_v.at[pl.ds(r * STRIDE + c * 104, HALF)]],
                    buf_v.at[pl.ds((2 * r + c) * HALF, HALF)],
                    gsem))
        for cp in copies:
            cp.wait()
        pltpu.async_copy(
            buf_v, out_hbm.at[pl.ds(row0 * HIST, SLAB * HIST)], wsem)

    def two(gg, _):
        body(2 * gg, 0)
        body(2 * gg + 1, 1)
        return 0

    lax.fori_loop(0, NSLAB // 2, two, 0)
    for p, wsem, buf_v in ((0, wsem0, buf_v0), (1, wsem1, buf_v1)):
        pltpu.make_async_copy(
            buf_v,
            out_hbm.at[pl.ds((base + (NSLAB - 2 + p) * SLAB) * HIST,
                             SLAB * HIST)],
            wsem).wait()


def _gather(pos_embeddings, rank_flat):
    mesh = plsc.VectorSubcoreMesh(
        core_axis_name="c", subcore_axis_name="s", num_cores=NC)
    f = functools.partial(
        pl.kernel,
        mesh=mesh,
        compiler_params=pltpu.CompilerParams(use_tc_tiling_on_sc=False),
        out_type=jax.ShapeDtypeStruct((BATCH * HIST, DIM), jnp.float32),
        scratch_types=[
            pltpu.VMEM((SLAB * HIST,), jnp.int32),
            pltpu.VMEM((SLAB * HIST,), jnp.int32),
            pltpu.VMEM((SLAB * HIST,), jnp.int32),
            pltpu.VMEM((SLAB * HIST,), jnp.int32),
            pltpu.VMEM((SLAB * STRIDE,), jnp.int32),
            pltpu.VMEM((SLAB * STRIDE,), jnp.int32),
            pltpu.VMEM_SHARED((NS * 2 * SLAB * STRIDE,), jnp.int32),
            pltpu.VMEM((SLAB * HIST, DIM), jnp.float32),
            pltpu.VMEM((SLAB * HIST, DIM), jnp.float32),
            pltpu.SemaphoreType.DMA,
            pltpu.SemaphoreType.DMA,
            pltpu.SemaphoreType.DMA,
        ],
    )(_gather_body)
    return f(pos_embeddings, rank_flat)


def kernel(ts, pos_embeddings):
    rank = _ranks(ts)
    out = _gather(pos_embeddings, rank.reshape(-1))
    return out.reshape(BATCH, HIST, DIM)


# speedup vs baseline: 1.1040x; 1.0015x over previous
import functools

import jax
import jax.numpy as jnp
from jax import lax
from jax.experimental import pallas as pl
from jax.experimental.pallas import tpu as pltpu
from jax.experimental.pallas import tpu_sc as plsc

BATCH = 16384
HIST = 200
DIM = 32
HALF = HIST // 2

BBLK = 64

NC, NS, L = 2, 16, 16
NW = NC * NS
ROWS_W = BATCH // NW
SLAB = 8
NSLAB = ROWS_W // SLAB
STRIDE = 208


def _rank_body(ts_ref, rank_ref):
    s = ts_ref[...]
    sk = s[:, :, None]
    sj = s[:, None, :]
    i2k = lax.broadcasted_iota(jnp.int32, (HIST, HIST), 0)
    i2j = lax.broadcasted_iota(jnp.int32, (HIST, HIST), 1)
    tri = (i2k < i2j)[None, :, :]
    lt = sk < sj
    le = sk <= sj
    cond = lt | (le & tri)
    cmp = jnp.where(cond, jnp.float32(1.0), jnp.float32(0.0))
    rank_ref[...] = jnp.sum(cmp, axis=1).astype(jnp.int32)


def _ranks(ts):
    return pl.pallas_call(
        _rank_body,
        grid=(BATCH // BBLK,),
        in_specs=[pl.BlockSpec((BBLK, HIST), lambda i: (i, 0))],
        out_specs=pl.BlockSpec((BBLK, HIST), lambda i: (i, 0)),
        out_shape=jax.ShapeDtypeStruct((BATCH, HIST), jnp.int32),
    )(ts)


def _gather_body(tab_hbm, rank_hbm, out_hbm,
                 rank_v0, rank_v1, idxb_v, val_v, ord_v0, ord_v1, ord_sp,
                 buf_v0, buf_v1, gsem, rsem, wsem0, wsem1):
    sid = lax.axis_index("s")
    wid = sid * NC + lax.axis_index("c")
    base = wid * ROWS_W
    sp_base = sid * (2 * SLAB * STRIDE)

    for j0 in range(0, SLAB * HIST, L):
        v = lax.broadcasted_iota(jnp.int32, (L,), 0) + j0
        val_v[pl.ds(j0, L)] = lax.rem(v, jnp.int32(HIST))

    def invert(g, p):
        rank_v = rank_v0 if p == 0 else rank_v1
        ord_v = ord_v0 if p == 0 else ord_v1
        pb = sp_base + p * (SLAB * STRIDE)
        for r in range(SLAB):
            for j0 in list(range(0, HIST - L + 1, L)) + [HIST - L]:
                rk = rank_v[pl.ds(r * HIST + j0, L)]
                hi = jnp.where(rk >= HALF, jnp.int32(4), jnp.int32(0))
                idxb_v[pl.ds(r * HIST + j0, L)] = rk + hi + (r * STRIDE + pb)
        pltpu.sync_copy(val_v, ord_sp.at[idxb_v])
        pltpu.sync_copy(ord_sp.at[pl.ds(pb, SLAB * STRIDE)], ord_v)

    def body(g, p):
        q = 1 - p
        row0 = base + g * SLAB
        ord_v = ord_v0 if p == 0 else ord_v1
        buf_v = buf_v0 if p == 0 else buf_v1
        wsem = wsem0 if p == 0 else wsem1
        rank_n = rank_v0 if q == 0 else rank_v1
        pre = None
        if True:
            pre = pltpu.async_copy(
                rank_hbm.at[pl.ds((row0 + SLAB) * HIST, SLAB * HIST)],
                rank_n, rsem)
        @pl.when(g >= 2)
        def _():
            pltpu.make_async_copy(
                buf_v,
                out_hbm.at[pl.ds((row0 - 2 * SLAB) * HIST, SLAB * HIST)],
                wsem).wait()
        copies = []
        for r in range(SLAB):
            for c in range(2):
                copies.append(pltpu.async_copy(
                    tab_hbm.at[ord_v.at[pl.ds(r * STRIDE + c * 104, HALF)]],
                    buf_v.at[pl.ds((2 * r + c) * HALF, HALF)],
                    gsem))
        pre.wait()
        invert(g + 1, q)
        for cp in copies:
            cp.wait()
        pltpu.async_copy(
            buf_v, out_hbm.at[pl.ds(row0 * HIST, SLAB * HIST)], wsem)

    def two(gg, _):
        body(2 * gg, 0)
        body(2 * gg + 1, 1)
        return 0

    pltpu.sync_copy(rank_hbm.at[pl.ds(base * HIST, SLAB * HIST)],
                    rank_v0)
    invert(0, 0)
    lax.fori_loop(0, NSLAB // 2 - 1, two, 0)
    body(NSLAB - 2, 0)
    g = NSLAB - 1
    row0 = base + g * SLAB
    buf_v, wsem, ord_v = buf_v1, wsem1, ord_v1
    pltpu.make_async_copy(
        buf_v, out_hbm.at[pl.ds((row0 - 2 * SLAB) * HIST, SLAB * HIST)],
        wsem).wait()
    copies = []
    for r in range(SLAB):
        for c in range(2):
            copies.append(pltpu.async_copy(
                tab_hbm.at[ord_v.at[pl.ds(r * STRIDE + c * 104, HALF)]],
                buf_v.at[pl.ds((2 * r + c) * HALF, HALF)],
                gsem))
    for cp in copies:
        cp.wait()
    pltpu.async_copy(buf_v, out_hbm.at[pl.ds(row0 * HIST, SLAB * HIST)], wsem)
    pltpu.make_async_copy(
        buf_v0,
        out_hbm.at[pl.ds((base + (NSLAB - 2) * SLAB) * HIST, SLAB * HIST)],
        wsem0).wait()
    pltpu.make_async_copy(
        buf_v1, out_hbm.at[pl.ds(row0 * HIST, SLAB * HIST)], wsem1).wait()


def _gather(pos_embeddings, rank_flat):
    mesh = plsc.VectorSubcoreMesh(
        core_axis_name="c", subcore_axis_name="s", num_cores=NC)
    f = functools.partial(
        pl.kernel,
        mesh=mesh,
        compiler_params=pltpu.CompilerParams(use_tc_tiling_on_sc=False),
        out_type=jax.ShapeDtypeStruct((BATCH * HIST, DIM), jnp.float32),
        scratch_types=[
            pltpu.VMEM((SLAB * HIST,), jnp.int32),
            pltpu.VMEM((SLAB * HIST,), jnp.int32),
            pltpu.VMEM((SLAB * HIST,), jnp.int32),
            pltpu.VMEM((SLAB * HIST,), jnp.int32),
            pltpu.VMEM((SLAB * STRIDE,), jnp.int32),
            pltpu.VMEM((SLAB * STRIDE,), jnp.int32),
            pltpu.VMEM_SHARED((NS * 2 * SLAB * STRIDE,), jnp.int32),
            pltpu.VMEM((SLAB * HIST, DIM), jnp.float32),
            pltpu.VMEM((SLAB * HIST, DIM), jnp.float32),
            pltpu.SemaphoreType.DMA,
            pltpu.SemaphoreType.DMA,
            pltpu.SemaphoreType.DMA,
            pltpu.SemaphoreType.DMA,
        ],
    )(_gather_body)
    return f(pos_embeddings, rank_flat)


def kernel(ts, pos_embeddings):
    rank = _ranks(ts)
    out = _gather(pos_embeddings, rank.reshape(-1))
    return out.reshape(BATCH, HIST, DIM)
